# baseline (speedup 1.0000x reference)
import functools

import jax
import jax.numpy as jnp
from jax import lax
from jax.experimental import pallas as pl
from jax.experimental.pallas import tpu as pltpu
from jax.experimental.pallas import tpu_sc as plsc

N = 9216
K = 8192
C = 256

TN = 512
TG = 2048
CH = 128
N_SUB = K // TG
N_CH = TG // CH

NH = N // 2
H_TILES = NH // TN


def _vq_body(x_ref, e_ref, out_ref, et_s, c_s):
    @pl.when(pl.program_id(0) == 0)
    def _():
        et = jnp.transpose(e_ref[...], (1, 0))
        et_s[...] = et
        c_s[...] = jnp.sum(et * et, axis=0, keepdims=True)

    et_ref = et_s
    c_ref = c_s
    x = x_ref[...]

    x2 = x + x
    a = jnp.sum(x * x, axis=1, keepdims=True)

    m2s = [
        lax.dot_general(
            x2, e_ref[pl.ds(g * TG, TG), :],
            dimension_numbers=(((1,), (1,)), ((), ())),
            preferred_element_type=jnp.float32)
        for g in range(N_SUB)
    ]

    bv = None
    bi = None
    for g in range(N_SUB):
        m2 = m2s[g]
        for j in range(N_CH):
            mj = lax.slice(m2, (0, j * CH), (TN, (j + 1) * CH))
            cj = c_ref[0, pl.ds(g * TG + j * CH, CH)]
            t = a - mj + cj
            base = g * TG + j * CH
            if bv is None:
                bv = t
                bi = jnp.zeros((TN, CH), jnp.int32)
            else:
                upd = t < bv
                bv = jnp.where(upd, t, bv)
                bi = jnp.where(upd, base, bi)

    gmin = jnp.min(bv, axis=1, keepdims=True)
    lane = lax.broadcasted_iota(jnp.int32, (TN, CH), 1)
    cand = jnp.where(bv == gmin, bi + lane, K)
    res = jnp.min(cand, axis=1, keepdims=True)
    out_ref[...] = jnp.reshape(res.T, (TN,))


def _vq_argmax(flat, embed, tile0):
    return pl.pallas_call(
        _vq_body,
        grid=(H_TILES,),
        in_specs=[
            pl.BlockSpec((TN, C), lambda n: (n + tile0, 0)),
            pl.BlockSpec((K, C), lambda n: (0, 0)),
        ],
        out_specs=pl.BlockSpec((TN,), lambda n: (n,)),
        out_shape=jax.ShapeDtypeStruct((NH,), jnp.int32),
        scratch_shapes=[
            pltpu.VMEM((C, K), jnp.float32),
            pltpu.VMEM((1, K), jnp.float32),
        ],
        compiler_params=pltpu.CompilerParams(
            dimension_semantics=("arbitrary",),
        ),
    )(flat, embed)


_info = plsc.get_sparse_core_info()
_NC, _NS = _info.num_cores, _info.num_subcores
_NW = _NC * _NS
_BPW = NH // _NW
_GCH = 72
_NCH = _BPW // _GCH


@functools.partial(
    pl.kernel,
    mesh=plsc.VectorSubcoreMesh(core_axis_name="c", subcore_axis_name="s"),
    out_type=jax.ShapeDtypeStruct((NH, C), jnp.float32),
    scratch_types=[
        pltpu.VMEM((_BPW,), jnp.int32),
        pltpu.VMEM((_BPW, C), jnp.float32),
        pltpu.SemaphoreType.DMA,
    ],
)
def _sc_gather(table_hbm, idx_hbm, out_hbm, idx_v, rows_v, sem):
    wid = lax.axis_index("s") * _NC + lax.axis_index("c")
    base = wid * _BPW
    pltpu.sync_copy(idx_hbm.at[pl.ds(base, _BPW)], idx_v)
    copies = []
    for j in range(_NCH):
        copies.append(pltpu.async_copy(
            table_hbm.at[idx_v.at[pl.ds(j * _GCH, _GCH)]],
            rows_v.at[pl.ds(j * _GCH, _GCH)],
            sem,
        ))
    for cp in copies:
        cp.wait()
    pltpu.sync_copy(rows_v, out_hbm.at[pl.ds(base, _BPW)])


def kernel(x, embed):
    Bb, Tt, Cc = x.shape
    flat = x.reshape(Bb * Tt, Cc)
    idx1 = _vq_argmax(flat, embed, 0)
    q1 = _sc_gather(embed, idx1)
    idx2 = _vq_argmax(flat, embed, H_TILES)
    q2 = _sc_gather(embed, idx2)
    return jnp.concatenate([q1, q2], axis=0).reshape(Bb, Tt, Cc)


# speedup vs baseline: 1.0111x; 1.0111x over previous
import functools

import jax
import jax.numpy as jnp
from jax import lax
from jax.experimental import pallas as pl
from jax.experimental.pallas import tpu as pltpu
from jax.experimental.pallas import tpu_sc as plsc

N = 9216
K = 8192
C = 256

TN = 512
TG = 2048
CH = 128
N_SUB = K // TG
N_CH = TG // CH

NH = N // 2
H_TILES = NH // TN


def _vq_body(x_ref, e_ref, out_ref, et_s, c_s):
    @pl.when(pl.program_id(0) == 0)
    def _():
        et = jnp.transpose(e_ref[...], (1, 0))
        et_s[...] = et
        c_s[...] = jnp.sum(et * et, axis=0, keepdims=True)

    et_ref = et_s
    c_ref = c_s
    x = x_ref[...]

    x2 = x + x
    a = jnp.sum(x * x, axis=1, keepdims=True)

    m2s = [
        jnp.dot(x2, et_ref[:, pl.ds(g * TG, TG)],
                preferred_element_type=jnp.float32)
        for g in range(N_SUB)
    ]

    bv = None
    bi = None
    for g in range(N_SUB):
        m2 = m2s[g]
        for j in range(N_CH):
            mj = lax.slice(m2, (0, j * CH), (TN, (j + 1) * CH))
            cj = c_ref[0, pl.ds(g * TG + j * CH, CH)]
            t = a - mj + cj
            base = g * TG + j * CH
            if bv is None:
                bv = t
                bi = jnp.zeros((TN, CH), jnp.int32)
            else:
                upd = t < bv
                bv = jnp.where(upd, t, bv)
                bi = jnp.where(upd, base, bi)

    gmin = jnp.min(bv, axis=1, keepdims=True)
    lane = lax.broadcasted_iota(jnp.int32, (TN, CH), 1)
    cand = jnp.where(bv == gmin, bi + lane, K)
    res = jnp.min(cand, axis=1, keepdims=True)
    out_ref[...] = jnp.reshape(res.T, (TN,))


def _vq_argmax(flat, embed, tile0):
    return pl.pallas_call(
        _vq_body,
        grid=(H_TILES,),
        in_specs=[
            pl.BlockSpec((TN, C), lambda n: (n + tile0, 0)),
            pl.BlockSpec((K, C), lambda n: (0, 0)),
        ],
        out_specs=pl.BlockSpec((TN,), lambda n: (n,)),
        out_shape=jax.ShapeDtypeStruct((NH,), jnp.int32),
        scratch_shapes=[
            pltpu.VMEM((C, K), jnp.float32),
            pltpu.VMEM((1, K), jnp.float32),
        ],
        compiler_params=pltpu.CompilerParams(
            dimension_semantics=("arbitrary",),
        ),
    )(flat, embed)


_info = plsc.get_sparse_core_info()
_NC, _NS = _info.num_cores, _info.num_subcores
_NW = _NC * _NS
_BPW = NH // _NW
_GCH = 72
_NCH = _BPW // _GCH


@functools.partial(
    pl.kernel,
    mesh=plsc.VectorSubcoreMesh(core_axis_name="c", subcore_axis_name="s"),
    out_type=jax.ShapeDtypeStruct((NH, C), jnp.float32),
    scratch_types=[
        pltpu.VMEM((_BPW,), jnp.int32),
        pltpu.VMEM((_BPW, C), jnp.float32),
        pltpu.SemaphoreType.DMA,
    ],
)
def _sc_gather(table_hbm, idx_hbm, out_hbm, idx_v, rows_v, sem):
    wid = lax.axis_index("s") * _NC + lax.axis_index("c")
    base = wid * _BPW
    pltpu.sync_copy(idx_hbm.at[pl.ds(base, _BPW)], idx_v)
    copies = []
    for j in range(_NCH):
        copies.append(pltpu.async_copy(
            table_hbm.at[idx_v.at[pl.ds(j * _GCH, _GCH)]],
            rows_v.at[pl.ds(j * _GCH, _GCH)],
            sem,
        ))
    for cp in copies:
        cp.wait()
    pltpu.sync_copy(rows_v, out_hbm.at[pl.ds(base, _BPW)])


def kernel(x, embed):
    Bb, Tt, Cc = x.shape
    flat = x.reshape(Bb * Tt, Cc)
    idx1 = _vq_argmax(flat, embed, 0)
    q1 = _sc_gather(embed, idx1)
    idx2 = _vq_argmax(flat, embed, H_TILES)
    q2 = _sc_gather(embed, idx2)
    return jnp.concatenate([q1, q2], axis=0).reshape(Bb, Tt, Cc)
